# baseline (speedup 1.0000x reference)
import jax
import jax.numpy as jnp
from jax import lax
from jax.experimental import pallas as pl
from jax.experimental.pallas import tpu as pltpu
from jax.experimental.pallas import tpu_sc as plsc

N_USERS = 100000
N_ITEMS = 900000
EMB = 32
REGS = 1e-5
B = 16384

NC = 2
NS = 16
NW = NC * NS
PB = B // NW
CHUNK = 128
NCH = PB // CHUNK


def _sc_body(table_hbm, u_idx_hbm, p_idx_hbm, n_idx_hbm,
             pd_hbm, sq_hbm,
             u_idx_v, p_idx_v, n_idx_v,
             u_rows, p_rows, n_rows,
             pd_v, sq_v, sem):
    wid = lax.axis_index("s") * NC + lax.axis_index("c")

    pltpu.sync_copy(u_idx_hbm.at[wid], u_idx_v)
    pltpu.sync_copy(p_idx_hbm.at[wid], p_idx_v)
    pltpu.sync_copy(n_idx_hbm.at[wid], n_idx_v)

    copies = []
    for idx_v, rows in ((u_idx_v, u_rows), (p_idx_v, p_rows), (n_idx_v, n_rows)):
        for j in range(NCH):
            copies.append(pltpu.async_copy(
                table_hbm.at[idx_v.at[j]],
                rows.at[pl.ds(j * CHUNK, CHUNK)],
                sem))
    for c in copies:
        c.wait()

    def body(i, sq):
        u0, u1 = plsc.unpack(u_rows[i, :], format=plsc.PackFormat.INTERLEAVED)
        p0, p1 = plsc.unpack(p_rows[i, :], format=plsc.PackFormat.INTERLEAVED)
        n0, n1 = plsc.unpack(n_rows[i, :], format=plsc.PackFormat.INTERLEAVED)
        pd_v[i, :] = u0 * (p0 - n0) + u1 * (p1 - n1)
        return (sq + u0 * u0 + u1 * u1 + p0 * p0 + p1 * p1
                + n0 * n0 + n1 * n1)

    sq = lax.fori_loop(0, PB, body, jnp.zeros((16,), jnp.float32))
    sq_v[...] = sq

    pltpu.sync_copy(pd_v, pd_hbm.at[wid])
    pltpu.sync_copy(sq_v, sq_hbm.at[wid])


def _sc_call(table, u_idx, p_idx, n_idx):
    mesh = plsc.VectorSubcoreMesh(core_axis_name="c", subcore_axis_name="s")
    return pl.kernel(
        _sc_body,
        out_type=(
            jax.ShapeDtypeStruct((NW, PB, 16), jnp.float32),
            jax.ShapeDtypeStruct((NW, 16), jnp.float32),
        ),
        mesh=mesh,
        compiler_params=pltpu.CompilerParams(
            use_tc_tiling_on_sc=False, needs_layout_passes=False),
        scratch_types=[
            pltpu.VMEM((NCH, CHUNK), jnp.int32),
            pltpu.VMEM((NCH, CHUNK), jnp.int32),
            pltpu.VMEM((NCH, CHUNK), jnp.int32),
            pltpu.VMEM((PB, EMB), jnp.bfloat16),
            pltpu.VMEM((PB, EMB), jnp.bfloat16),
            pltpu.VMEM((PB, EMB), jnp.bfloat16),
            pltpu.VMEM((PB, 16), jnp.float32),
            pltpu.VMEM((16,), jnp.float32),
            pltpu.SemaphoreType.DMA,
        ],
    )(table, u_idx, p_idx, n_idx)


def _tc_epilogue_body(pd_ref, sq_ref, bpr_ref, reg_ref):
    d = jnp.sum(pd_ref[...], axis=1, keepdims=True)
    logsig = -jnp.log1p(jnp.exp(-d))
    bpr_ref[...] = jnp.full((1, 1), -jnp.mean(logsig), jnp.float32)
    reg_ref[...] = jnp.full((1, 1), REGS * 0.5 * jnp.sum(sq_ref[...]),
                            jnp.float32)


def _tc_epilogue(pd, sq):
    return pl.pallas_call(
        _tc_epilogue_body,
        out_shape=(
            jax.ShapeDtypeStruct((1, 1), jnp.float32),
            jax.ShapeDtypeStruct((1, 1), jnp.float32),
        ),
    )(pd, sq)


@jax.jit
def kernel(user, pos_item, neg_item, table):
    u_idx = user.reshape(NW, NCH, CHUNK)
    p_idx = pos_item.reshape(NW, NCH, CHUNK)
    n_idx = neg_item.reshape(NW, NCH, CHUNK)
    pd, sq = _sc_call(table.astype(jnp.bfloat16), u_idx, p_idx, n_idx)
    bpr, reg = _tc_epilogue(pd.reshape(B, 16), sq)
    return (bpr.reshape(()), reg.reshape(()))


# speedup vs baseline: 1.4688x; 1.4688x over previous
import jax
import jax.numpy as jnp
from jax import lax
from jax.experimental import pallas as pl
from jax.experimental.pallas import tpu as pltpu
from jax.experimental.pallas import tpu_sc as plsc

N_USERS = 100000
N_ITEMS = 900000
EMB = 32
REGS = 1e-5
B = 16384

NC = 2
NS = 16
NW = NC * NS
PB = B // NW
CHUNK = 128
NCH = PB // CHUNK


def _sc_body(table_hbm, u_idx_hbm, p_idx_hbm, n_idx_hbm,
             pd_hbm, sq_hbm,
             u_idx_v, p_idx_v, n_idx_v,
             u_rows, p_rows, n_rows,
             pd_v, sq_v, sem):
    wid = lax.axis_index("s") * NC + lax.axis_index("c")

    pltpu.sync_copy(u_idx_hbm.at[wid], u_idx_v)
    pltpu.sync_copy(p_idx_hbm.at[wid], p_idx_v)
    pltpu.sync_copy(n_idx_hbm.at[wid], n_idx_v)

    copies = []
    for idx_v, rows in ((u_idx_v, u_rows), (p_idx_v, p_rows), (n_idx_v, n_rows)):
        for j in range(NCH):
            copies.append(pltpu.async_copy(
                table_hbm.at[idx_v.at[j]],
                rows.at[pl.ds(j * CHUNK, CHUNK)],
                sem))
    for c in copies:
        c.wait()

    def body(i, sq):
        u0 = u_rows[i, pl.ds(0, 16)]
        u1 = u_rows[i, pl.ds(16, 16)]
        p0 = p_rows[i, pl.ds(0, 16)]
        p1 = p_rows[i, pl.ds(16, 16)]
        n0 = n_rows[i, pl.ds(0, 16)]
        n1 = n_rows[i, pl.ds(16, 16)]
        pd_v[i, :] = u0 * (p0 - n0) + u1 * (p1 - n1)
        return (sq + u0 * u0 + u1 * u1 + p0 * p0 + p1 * p1
                + n0 * n0 + n1 * n1)

    sq = lax.fori_loop(0, PB, body, jnp.zeros((16,), jnp.float32))
    sq_v[...] = sq

    pltpu.sync_copy(pd_v, pd_hbm.at[wid])
    pltpu.sync_copy(sq_v, sq_hbm.at[wid])


def _sc_call(table, u_idx, p_idx, n_idx):
    mesh = plsc.VectorSubcoreMesh(core_axis_name="c", subcore_axis_name="s")
    return pl.kernel(
        _sc_body,
        out_type=(
            jax.ShapeDtypeStruct((NW, PB, 16), jnp.float32),
            jax.ShapeDtypeStruct((NW, 16), jnp.float32),
        ),
        mesh=mesh,
        compiler_params=pltpu.CompilerParams(use_tc_tiling_on_sc=False),
        scratch_types=[
            pltpu.VMEM((NCH, CHUNK), jnp.int32),
            pltpu.VMEM((NCH, CHUNK), jnp.int32),
            pltpu.VMEM((NCH, CHUNK), jnp.int32),
            pltpu.VMEM((PB, EMB), jnp.float32),
            pltpu.VMEM((PB, EMB), jnp.float32),
            pltpu.VMEM((PB, EMB), jnp.float32),
            pltpu.VMEM((PB, 16), jnp.float32),
            pltpu.VMEM((16,), jnp.float32),
            pltpu.SemaphoreType.DMA,
        ],
    )(table, u_idx, p_idx, n_idx)


def _tc_epilogue_body(pd_ref, sq_ref, bpr_ref, reg_ref):
    d = jnp.sum(pd_ref[...], axis=1, keepdims=True)
    logsig = -jnp.log1p(jnp.exp(-d))
    bpr_ref[...] = jnp.full((1, 1), -jnp.mean(logsig), jnp.float32)
    reg_ref[...] = jnp.full((1, 1), REGS * 0.5 * jnp.sum(sq_ref[...]),
                            jnp.float32)


def _tc_epilogue(pd, sq):
    return pl.pallas_call(
        _tc_epilogue_body,
        out_shape=(
            jax.ShapeDtypeStruct((1, 1), jnp.float32),
            jax.ShapeDtypeStruct((1, 1), jnp.float32),
        ),
    )(pd, sq)


@jax.jit
def kernel(user, pos_item, neg_item, table):
    u_idx = user.reshape(NW, NCH, CHUNK)
    p_idx = pos_item.reshape(NW, NCH, CHUNK)
    n_idx = neg_item.reshape(NW, NCH, CHUNK)
    pd, sq = _sc_call(table, u_idx, p_idx, n_idx)
    bpr, reg = _tc_epilogue(pd.reshape(B, 16), sq)
    return (bpr.reshape(()), reg.reshape(()))
